# baseline (speedup 1.0000x reference)
import functools

import jax
import jax.numpy as jnp
import numpy as np
from jax.experimental import pallas as pl
from jax.experimental.pallas import tpu as pltpu

T = 8
M = 1.0
LAM = 0.01
BLK = 128


def _tc_kernel(v_ref, vh_ref, g_ref, f_ref, neg_ref, mask_ref, out_ref, acc):
    pid = pl.program_id(0)
    nblk = pl.num_programs(0)

    @pl.when(pid == 0)
    def _init():
        f = f_ref[...]
        gram = jax.lax.dot_general(f, f, (((1,), (1,)), ((), ())),
                                   preferred_element_type=jnp.float32)
        k = gram.shape[0]
        rows = jax.lax.broadcasted_iota(jnp.int32, gram.shape, 0)
        cols = jax.lax.broadcasted_iota(jnp.int32, gram.shape, 1)
        tr = jnp.sum(jnp.where(rows == cols, gram, 0.0))
        acc[0] = 0.0
        acc[1] = 0.0
        acc[2] = jnp.sum(gram * gram) - 2.0 * tr + float(k)

    vhat = vh_ref[...]
    v = v_ref[...]
    vh2 = jnp.sum(vhat * vhat, axis=1)
    td = jnp.sqrt(jnp.sum((vhat - v) ** 2, axis=1) + 1e-8)

    neg = neg_ref[...]
    nn2 = jnp.sum(neg * neg, axis=1)
    ndots = jax.lax.dot_general(vhat, neg, (((1,), (1,)), ((), ())),
                                preferred_element_type=jnp.float32)
    nd = jnp.sqrt(jnp.maximum(vh2[:, None] - 2.0 * ndots + nn2[None, :], 0.0)
                  + 1e-8)
    c = jnp.mean(jnp.maximum(1.0 + td[:, None] - nd, 0.0), axis=1)

    f = f_ref[...]
    fn2 = jnp.sum(f * f, axis=1)
    dots = jax.lax.dot_general(vhat, f, (((1,), (1,)), ((), ())),
                               preferred_element_type=jnp.float32)
    h = fn2[None, :] - 2.0 * dots

    g = g_ref[...]
    kk = g.shape[1]
    gi = jax.lax.bitcast_convert_type(g, jnp.int32)
    col = jax.lax.broadcasted_iota(jnp.int32, g.shape, 1)
    keys = (gi & np.int32(~511)) | col
    gts, hts = [], []
    for _ in range(T):
        kmin = jnp.min(keys, axis=1)
        sel = keys == kmin[:, None]
        hts.append(jnp.sum(jnp.where(sel, h, 0.0), axis=1))
        gts.append(jax.lax.bitcast_convert_type(kmin & np.int32(~511),
                                                jnp.float32))
        keys = jnp.where(sel, np.int32(2**31 - 1), keys)
    gt = jnp.stack(gts, axis=1)
    ht = jnp.stack(hts, axis=1)
    gn = gt / (jnp.sum(gt, axis=1, keepdims=True) + 1e-10)
    mt = M * (1.0 - gn) ** 2
    dist = jnp.sqrt(jnp.maximum(vh2[:, None] + ht, 0.0) + 1e-8)
    jt = jnp.mean(jnp.maximum(mt + td[:, None] - dist, 0.0), axis=1)

    mask = mask_ref[0, 0, :]
    acc[0] += jnp.sum(mask * (c + jt))
    acc[1] += jnp.sum(mask)

    @pl.when(pid == nblk - 1)
    def _fin():
        val = acc[0] / jnp.maximum(acc[1], 1.0) + LAM * acc[2]
        out_ref[...] = jnp.broadcast_to(val, (1, 1))


@functools.partial(jax.jit, static_argnames=())
def kernel(v, vhat, d, g, F, negatives, mask):
    del d
    B, D = v.shape
    K = F.shape[0]
    N = negatives.shape[0]
    nblk = B // BLK
    maskf = mask.astype(jnp.float32).reshape(nblk, 1, BLK)

    out = pl.pallas_call(
        _tc_kernel,
        grid=(nblk,),
        in_specs=[
            pl.BlockSpec((BLK, D), lambda i: (i, 0)),
            pl.BlockSpec((BLK, D), lambda i: (i, 0)),
            pl.BlockSpec((BLK, K), lambda i: (i, 0)),
            pl.BlockSpec((K, D), lambda i: (0, 0)),
            pl.BlockSpec((N, D), lambda i: (0, 0)),
            pl.BlockSpec((1, 1, BLK), lambda i: (i, 0, 0)),
        ],
        out_specs=pl.BlockSpec((1, 1), lambda i: (0, 0)),
        out_shape=jax.ShapeDtypeStruct((1, 1), jnp.float32),
        scratch_shapes=[pltpu.SMEM((3,), jnp.float32)],
    )(v, vhat, g, F, negatives, maskf)
    return out.reshape(())


# speedup vs baseline: 10.1822x; 10.1822x over previous
import functools

import jax
import jax.numpy as jnp
import numpy as np
from jax.experimental import pallas as pl
from jax.experimental.pallas import tpu as pltpu

T = 8
M = 1.0
LAM = 0.01
BLK = 128


def _tc_kernel(v_ref, vh_ref, g_ref, f_ref, neg_ref, mask_ref, out_ref, acc):
    pid = pl.program_id(0)
    nblk = pl.num_programs(0)

    @pl.when(pid == 0)
    def _init():
        f = f_ref[...]
        gram = jax.lax.dot_general(f, f, (((1,), (1,)), ((), ())),
                                   preferred_element_type=jnp.float32)
        k = gram.shape[0]
        rows = jax.lax.broadcasted_iota(jnp.int32, gram.shape, 0)
        cols = jax.lax.broadcasted_iota(jnp.int32, gram.shape, 1)
        tr = jnp.sum(jnp.where(rows == cols, gram, 0.0))
        acc[0] = 0.0
        acc[1] = 0.0
        acc[2] = jnp.sum(gram * gram) - 2.0 * tr + float(k)

    vhat = vh_ref[...]
    v = v_ref[...]
    vh2 = jnp.sum(vhat * vhat, axis=1)
    td = jnp.sqrt(jnp.sum((vhat - v) ** 2, axis=1) + 1e-8)

    ones_row = jnp.ones((8, v.shape[1]), jnp.float32)
    neg = neg_ref[...]
    nn2 = jax.lax.dot_general(ones_row, neg * neg, (((1,), (1,)), ((), ())),
                              preferred_element_type=jnp.float32)[0:1, :]
    ndots = jax.lax.dot_general(vhat, neg, (((1,), (1,)), ((), ())),
                                preferred_element_type=jnp.float32)
    nd = jnp.sqrt(jnp.maximum(vh2[:, None] - 2.0 * ndots + nn2, 0.0)
                  + 1e-8)
    c = jnp.mean(jnp.maximum(1.0 + td[:, None] - nd, 0.0), axis=1)

    f = f_ref[...]
    fn2 = jax.lax.dot_general(ones_row, f * f, (((1,), (1,)), ((), ())),
                              preferred_element_type=jnp.float32)[0:1, :]
    dots = jax.lax.dot_general(vhat, f, (((1,), (1,)), ((), ())),
                               preferred_element_type=jnp.float32)
    h = fn2 - 2.0 * dots

    g = g_ref[...]
    kk = g.shape[1]
    gi = jax.lax.bitcast_convert_type(g, jnp.int32)
    col = jax.lax.broadcasted_iota(jnp.int32, g.shape, 1)
    keys = (gi & np.int32(~511)) | col
    gts, hts = [], []
    for _ in range(T):
        kmin = jnp.min(keys, axis=1)
        sel = keys == kmin[:, None]
        hts.append(jnp.sum(jnp.where(sel, h, 0.0), axis=1))
        gts.append(jax.lax.bitcast_convert_type(kmin & np.int32(~511),
                                                jnp.float32))
        keys = jnp.where(sel, np.int32(2**31 - 1), keys)
    gt = jnp.stack(gts, axis=1)
    ht = jnp.stack(hts, axis=1)
    gn = gt / (jnp.sum(gt, axis=1, keepdims=True) + 1e-10)
    mt = M * (1.0 - gn) ** 2
    dist = jnp.sqrt(jnp.maximum(vh2[:, None] + ht, 0.0) + 1e-8)
    jt = jnp.mean(jnp.maximum(mt + td[:, None] - dist, 0.0), axis=1)

    mask = mask_ref[0, 0, :]
    acc[0] += jnp.sum(mask * (c + jt))
    acc[1] += jnp.sum(mask)

    @pl.when(pid == nblk - 1)
    def _fin():
        val = acc[0] / jnp.maximum(acc[1], 1.0) + LAM * acc[2]
        out_ref[...] = jnp.broadcast_to(val, (1, 1))


@functools.partial(jax.jit, static_argnames=())
def kernel(v, vhat, d, g, F, negatives, mask):
    del d
    B, D = v.shape
    K = F.shape[0]
    N = negatives.shape[0]
    nblk = B // BLK
    maskf = mask.astype(jnp.float32).reshape(nblk, 1, BLK)

    out = pl.pallas_call(
        _tc_kernel,
        grid=(nblk,),
        in_specs=[
            pl.BlockSpec((BLK, D), lambda i: (i, 0)),
            pl.BlockSpec((BLK, D), lambda i: (i, 0)),
            pl.BlockSpec((BLK, K), lambda i: (i, 0)),
            pl.BlockSpec((K, D), lambda i: (0, 0)),
            pl.BlockSpec((N, D), lambda i: (0, 0)),
            pl.BlockSpec((1, 1, BLK), lambda i: (i, 0, 0)),
        ],
        out_specs=pl.BlockSpec((1, 1), lambda i: (0, 0)),
        out_shape=jax.ShapeDtypeStruct((1, 1), jnp.float32),
        scratch_shapes=[pltpu.SMEM((3,), jnp.float32)],
    )(v, vhat, g, F, negatives, maskf)
    return out.reshape(())


# speedup vs baseline: 19.7857x; 1.9432x over previous
import functools

import jax
import jax.numpy as jnp
import numpy as np
from jax.experimental import pallas as pl
from jax.experimental.pallas import tpu as pltpu

T = 8
M = 1.0
LAM = 0.01
BLK = 512


def _tc_kernel(v_ref, vh_ref, g_ref, f_ref, neg_ref, mask_ref, out_ref, acc):
    pid = pl.program_id(0)
    nblk = pl.num_programs(0)

    @pl.when(pid == 0)
    def _init():
        f = f_ref[...]
        gram = jax.lax.dot_general(f, f, (((1,), (1,)), ((), ())),
                                   preferred_element_type=jnp.float32)
        k = gram.shape[0]
        rows = jax.lax.broadcasted_iota(jnp.int32, gram.shape, 0)
        cols = jax.lax.broadcasted_iota(jnp.int32, gram.shape, 1)
        tr = jnp.sum(jnp.where(rows == cols, gram, 0.0))
        acc[0] = 0.0
        acc[1] = 0.0
        acc[2] = jnp.sum(gram * gram) - 2.0 * tr + float(k)

    vhat = vh_ref[...]
    v = v_ref[...]
    vh2 = jnp.sum(vhat * vhat, axis=1)
    td = jnp.sqrt(jnp.sum((vhat - v) ** 2, axis=1) + 1e-8)

    ones_row = jnp.ones((8, v.shape[1]), jnp.float32)
    neg = neg_ref[...]
    nn2 = jax.lax.dot_general(ones_row, neg * neg, (((1,), (1,)), ((), ())),
                              preferred_element_type=jnp.float32)[0:1, :]
    ndots = jax.lax.dot_general(vhat, neg, (((1,), (1,)), ((), ())),
                                preferred_element_type=jnp.float32)
    nd = jnp.sqrt(jnp.maximum(vh2[:, None] - 2.0 * ndots + nn2, 0.0)
                  + 1e-8)
    c = jnp.mean(jnp.maximum(1.0 + td[:, None] - nd, 0.0), axis=1)

    f = f_ref[...]
    fn2 = jax.lax.dot_general(ones_row, f * f, (((1,), (1,)), ((), ())),
                              preferred_element_type=jnp.float32)[0:1, :]
    dots = jax.lax.dot_general(vhat, f, (((1,), (1,)), ((), ())),
                               preferred_element_type=jnp.float32)
    h = fn2 - 2.0 * dots

    g = g_ref[...]
    kk = g.shape[1]
    gi = jax.lax.bitcast_convert_type(g, jnp.int32)
    col = jax.lax.broadcasted_iota(jnp.int32, g.shape, 1)
    keys = (gi & np.int32(~511)) | col
    gts, hts = [], []
    for _ in range(T):
        kmin = jnp.min(keys, axis=1)
        sel = keys == kmin[:, None]
        hts.append(jnp.sum(jnp.where(sel, h, 0.0), axis=1))
        gts.append(jax.lax.bitcast_convert_type(kmin & np.int32(~511),
                                                jnp.float32))
        keys = jnp.where(sel, np.int32(2**31 - 1), keys)
    gt = jnp.stack(gts, axis=1)
    ht = jnp.stack(hts, axis=1)
    gn = gt / (jnp.sum(gt, axis=1, keepdims=True) + 1e-10)
    mt = M * (1.0 - gn) ** 2
    dist = jnp.sqrt(jnp.maximum(vh2[:, None] + ht, 0.0) + 1e-8)
    jt = jnp.mean(jnp.maximum(mt + td[:, None] - dist, 0.0), axis=1)

    mask = mask_ref[0, 0, :]
    acc[0] += jnp.sum(mask * (c + jt))
    acc[1] += jnp.sum(mask)

    @pl.when(pid == nblk - 1)
    def _fin():
        val = acc[0] / jnp.maximum(acc[1], 1.0) + LAM * acc[2]
        out_ref[...] = jnp.broadcast_to(val, (1, 1))


@functools.partial(jax.jit, static_argnames=())
def kernel(v, vhat, d, g, F, negatives, mask):
    del d
    B, D = v.shape
    K = F.shape[0]
    N = negatives.shape[0]
    nblk = B // BLK
    maskf = mask.astype(jnp.float32).reshape(nblk, 1, BLK)

    out = pl.pallas_call(
        _tc_kernel,
        grid=(nblk,),
        in_specs=[
            pl.BlockSpec((BLK, D), lambda i: (i, 0)),
            pl.BlockSpec((BLK, D), lambda i: (i, 0)),
            pl.BlockSpec((BLK, K), lambda i: (i, 0)),
            pl.BlockSpec((K, D), lambda i: (0, 0)),
            pl.BlockSpec((N, D), lambda i: (0, 0)),
            pl.BlockSpec((1, 1, BLK), lambda i: (i, 0, 0)),
        ],
        out_specs=pl.BlockSpec((1, 1), lambda i: (0, 0)),
        out_shape=jax.ShapeDtypeStruct((1, 1), jnp.float32),
        scratch_shapes=[pltpu.SMEM((3,), jnp.float32)],
    )(v, vhat, g, F, negatives, maskf)
    return out.reshape(())
